# baseline (speedup 1.0000x reference)
import functools

import jax
import jax.numpy as jnp
from jax import lax
from jax.experimental import pallas as pl
from jax.experimental.pallas import tpu as pltpu, tpu_sc as plsc

B, N, K = 4, 8192, 16
C = 64
B2 = 2
NCHAIN = B // B2
BN2 = B2 * N
NW = 32
PW = BN2 // NW
WPB = N // PW
CH = 8
NCH = PW // CH
NRING = 8
NB = 2048
LG = 32


def _lrelu(x):
    return jnp.where(x >= 0, x, 0.1 * x)


def _dot(a, b, dims):
    return lax.dot_general(a, b, (dims, ((), ())),
                           preferred_element_type=jnp.float32)


def _stage_a_body(xyz_ref, feat_ref, wx_ref, wf_ref, h_ref):
    x = xyz_ref[0]
    f = feat_ref[0]
    h = (_dot(x, wx_ref[...], ((0,), (0,)))
         + _dot(f, wf_ref[...], ((0,), (0,))))
    h_ref[0] = h.astype(jnp.bfloat16)


def _stage_b_body(m_ref, xyz_ref, w1x_ref, b1_ref, w2x_ref, w2f_ref, h2_ref):
    x = xyz_ref[0]
    p1 = _dot(x, w1x_ref[...], ((0,), (0,)))
    f1 = _lrelu(m_ref[0].astype(jnp.float32) - p1 + b1_ref[...])
    h2 = (_dot(x, w2x_ref[...], ((0,), (0,)))
          + _dot(f1, w2f_ref[...], ((1,), (0,))))
    h2_ref[0] = h2.astype(jnp.bfloat16)


def _stage_c_body(m_ref, xyz_ref, w2x_ref, b2_ref, wm1_ref, bm1_ref,
                  wm2_ref, bm2_ref, wl_ref, bl_ref, ff_ref, flow_ref):
    x = xyz_ref[0]
    p2 = _dot(x, w2x_ref[...], ((0,), (0,)))
    f2 = _lrelu(m_ref[0].astype(jnp.float32) - p2 + b2_ref[...])
    f3 = _lrelu(_dot(f2, wm1_ref[...], ((1,), (0,))) + bm1_ref[...])
    ff = _lrelu(_dot(wm2_ref[...], f3, ((0,), (1,))) + bm2_ref[...])
    ff_ref[0] = ff
    flow_ref[0] = _dot(wl_ref[...], ff, ((0,), (0,))) + bl_ref[...]


def _cspec(c, nb, b0):
    return pl.BlockSpec((1, c, nb), lambda b, j: (b0 + b, 0, j))


def _rspec(nb, c, b0=0):
    return pl.BlockSpec((1, nb, c), lambda b, j: (b0 + b, j, 0))


def _fspec(shape):
    return pl.BlockSpec(shape, lambda b, j: tuple(0 for _ in shape))


_GRID = (B2, N // NB)


def _make_stage_a(b0):
    return pl.pallas_call(
        _stage_a_body,
        grid=_GRID,
        in_specs=[_cspec(3, NB, b0), _cspec(C, NB, b0),
                  _fspec((3, C)), _fspec((C, C))],
        out_specs=_rspec(NB, C),
        out_shape=jax.ShapeDtypeStruct((B2, N, C), jnp.bfloat16),
    )


def _make_stage_b(b0):
    return pl.pallas_call(
        _stage_b_body,
        grid=_GRID,
        in_specs=[_rspec(NB, C), _cspec(3, NB, b0), _fspec((3, C)),
                  _fspec((1, C)), _fspec((3, C)), _fspec((C, C))],
        out_specs=_rspec(NB, C),
        out_shape=jax.ShapeDtypeStruct((B2, N, C), jnp.bfloat16),
    )


def _make_stage_c(b0):
    return pl.pallas_call(
        _stage_c_body,
        grid=_GRID,
        in_specs=[_rspec(NB, C), _cspec(3, NB, b0), _fspec((3, C)),
                  _fspec((1, C)), _fspec((C, C)), _fspec((1, C)),
                  _fspec((C, C)), _fspec((C, 1)), _fspec((C, 3)),
                  _fspec((3, 1))],
        out_specs=[pl.BlockSpec((1, C, NB), lambda b, j: (b, 0, j)),
                   pl.BlockSpec((1, 3, NB), lambda b, j: (b, 0, j))],
        out_shape=[jax.ShapeDtypeStruct((B2, C, N), jnp.float32),
                   jax.ShapeDtypeStruct((B2, 3, N), jnp.float32)],
    )


_stages_a = [_make_stage_a(c * B2) for c in range(NCHAIN)]
_stages_b = [_make_stage_b(c * B2) for c in range(NCHAIN)]
_stages_c = [_make_stage_c(c * B2) for c in range(NCHAIN)]


_SC_MESH = plsc.VectorSubcoreMesh(core_axis_name="c", subcore_axis_name="s")


@functools.partial(
    pl.kernel,
    out_type=jax.ShapeDtypeStruct((BN2, C), jnp.bfloat16),
    mesh=_SC_MESH,
    compiler_params=pltpu.CompilerParams(use_tc_tiling_on_sc=False),
    scratch_types=[
        pltpu.VMEM((PW * K,), jnp.int32),
        *[pltpu.VMEM((CH * K, C), jnp.bfloat16) for _ in range(NRING)],
        pltpu.VMEM((PW, C), jnp.bfloat16),
        *[pltpu.SemaphoreType.DMA for _ in range(NRING)],
    ],
)
def _gather_max(h_hbm, idx_hbm, out_hbm, idx_v, *rest):
    rows = rest[:NRING]
    out_v = rest[NRING]
    sems = rest[NRING + 1:]
    wid = lax.axis_index("s") * 2 + lax.axis_index("c")
    base = wid * PW
    src = h_hbm.at[pl.ds((wid // WPB) * N, N)]
    pltpu.sync_copy(idx_hbm.at[pl.ds(base * K, PW * K)], idx_v)

    def _issue(c, r):
        pltpu.async_copy(src.at[idx_v.at[pl.ds(c * (CH * K), CH * K)]],
                         rows[r], sems[r])

    def _wait(c, r):
        pltpu.make_async_copy(
            src.at[idx_v.at[pl.ds(c * (CH * K), CH * K)]],
            rows[r], sems[r]).wait()

    def _compute(c, r):
        buf = rows[r]
        for i in range(CH):
            for cg in range(C // LG):
                vals = [buf[i * K + k, pl.ds(cg * LG, LG)] for k in range(K)]
                while len(vals) > 1:
                    nxt = [jnp.maximum(vals[j], vals[j + 1])
                           for j in range(0, len(vals) - 1, 2)]
                    if len(vals) % 2:
                        nxt.append(vals[-1])
                    vals = nxt
                out_v[c * CH + i, pl.ds(cg * LG, LG)] = vals[0]

    for r in range(NRING - 1):
        _issue(r, r)

    def _body(g, carry):
        c0 = g * NRING
        for r in range(NRING):
            nxt = c0 + r + NRING - 1

            @pl.when(nxt < NCH)
            def _():
                _issue(nxt, (r + NRING - 1) % NRING)

            _wait(c0 + r, r)
            _compute(c0 + r, r)
        return carry

    lax.fori_loop(0, NCH // NRING, _body, 0)

    pltpu.sync_copy(out_v, out_hbm.at[pl.ds(base, PW)])


def kernel(xyz, feat, knn_indices, W1, b1, W2, b2, Wm1, bm1, Wm2, bm2, Wl, bl):
    idx = knn_indices.astype(jnp.int32)

    W1x = W1[:3]
    W1f = W1[3:]
    W2x = W2[:3]
    W2f = W2[3:]

    ffs, fls = [], []
    for c in range(NCHAIN):
        idxf = idx[c * B2:(c + 1) * B2].reshape(-1)
        h1 = _stages_a[c](xyz, feat, W1x, W1f)
        m1 = _gather_max(h1.reshape(BN2, C), idxf)
        h2 = _stages_b[c](m1.reshape(B2, N, C), xyz, W1x, b1[None, :],
                          W2x, W2f)
        m2 = _gather_max(h2.reshape(BN2, C), idxf)
        ff, fl = _stages_c[c](m2.reshape(B2, N, C), xyz, W2x, b2[None, :],
                              Wm1, bm1[None, :], Wm2, bm2[:, None],
                              Wl, bl[:, None])
        ffs.append(ff)
        fls.append(fl)
    return (jnp.concatenate(ffs, axis=0), jnp.concatenate(fls, axis=0))


# speedup vs baseline: 1.0940x; 1.0940x over previous
import functools

import jax
import jax.numpy as jnp
from jax import lax
from jax.experimental import pallas as pl
from jax.experimental.pallas import tpu as pltpu, tpu_sc as plsc

B, N, K = 4, 8192, 16
C = 64
B2 = 2
NCHAIN = B // B2
BN2 = B2 * N
NW = 32
PW = BN2 // NW
WPB = N // PW
CH = 8
NCH = PW // CH
NRING = 4
NB = 2048
LG = 32


def _lrelu(x):
    return jnp.where(x >= 0, x, 0.1 * x)


def _dot(a, b, dims):
    return lax.dot_general(a, b, (dims, ((), ())),
                           preferred_element_type=jnp.float32)


def _stage_a_body(xyz_ref, feat_ref, wx_ref, wf_ref, h_ref):
    x = xyz_ref[0]
    f = feat_ref[0]
    h = (_dot(x, wx_ref[...], ((0,), (0,)))
         + _dot(f, wf_ref[...], ((0,), (0,))))
    h_ref[0] = h.astype(jnp.bfloat16)


def _stage_b_body(m_ref, xyz_ref, w1x_ref, b1_ref, w2x_ref, w2f_ref, h2_ref):
    x = xyz_ref[0]
    p1 = _dot(x, w1x_ref[...], ((0,), (0,)))
    f1 = _lrelu(m_ref[0].astype(jnp.float32) - p1 + b1_ref[...])
    h2 = (_dot(x, w2x_ref[...], ((0,), (0,)))
          + _dot(f1, w2f_ref[...], ((1,), (0,))))
    h2_ref[0] = h2.astype(jnp.bfloat16)


def _stage_c_body(m_ref, xyz_ref, w2x_ref, b2_ref, wm1_ref, bm1_ref,
                  wm2_ref, bm2_ref, wl_ref, bl_ref, ff_ref, flow_ref):
    x = xyz_ref[0]
    p2 = _dot(x, w2x_ref[...], ((0,), (0,)))
    f2 = _lrelu(m_ref[0].astype(jnp.float32) - p2 + b2_ref[...])
    f3 = _lrelu(_dot(f2, wm1_ref[...], ((1,), (0,))) + bm1_ref[...])
    ff = _lrelu(_dot(wm2_ref[...], f3, ((0,), (1,))) + bm2_ref[...])
    ff_ref[0] = ff
    flow_ref[0] = _dot(wl_ref[...], ff, ((0,), (0,))) + bl_ref[...]


def _cspec(c, nb, b0):
    return pl.BlockSpec((1, c, nb), lambda b, j: (b0 + b, 0, j))


def _rspec(nb, c, b0=0):
    return pl.BlockSpec((1, nb, c), lambda b, j: (b0 + b, j, 0))


def _fspec(shape):
    return pl.BlockSpec(shape, lambda b, j: tuple(0 for _ in shape))


_GRID = (B2, N // NB)


def _make_stage_a(b0):
    return pl.pallas_call(
        _stage_a_body,
        grid=_GRID,
        in_specs=[_cspec(3, NB, b0), _cspec(C, NB, b0),
                  _fspec((3, C)), _fspec((C, C))],
        out_specs=_rspec(NB, C),
        out_shape=jax.ShapeDtypeStruct((B2, N, C), jnp.bfloat16),
    )


def _make_stage_b(b0):
    return pl.pallas_call(
        _stage_b_body,
        grid=_GRID,
        in_specs=[_rspec(NB, C), _cspec(3, NB, b0), _fspec((3, C)),
                  _fspec((1, C)), _fspec((3, C)), _fspec((C, C))],
        out_specs=_rspec(NB, C),
        out_shape=jax.ShapeDtypeStruct((B2, N, C), jnp.bfloat16),
    )


def _make_stage_c(b0):
    return pl.pallas_call(
        _stage_c_body,
        grid=_GRID,
        in_specs=[_rspec(NB, C), _cspec(3, NB, b0), _fspec((3, C)),
                  _fspec((1, C)), _fspec((C, C)), _fspec((1, C)),
                  _fspec((C, C)), _fspec((C, 1)), _fspec((C, 3)),
                  _fspec((3, 1))],
        out_specs=[pl.BlockSpec((1, C, NB), lambda b, j: (b, 0, j)),
                   pl.BlockSpec((1, 3, NB), lambda b, j: (b, 0, j))],
        out_shape=[jax.ShapeDtypeStruct((B2, C, N), jnp.float32),
                   jax.ShapeDtypeStruct((B2, 3, N), jnp.float32)],
    )


_stages_a = [_make_stage_a(c * B2) for c in range(NCHAIN)]
_stages_b = [_make_stage_b(c * B2) for c in range(NCHAIN)]
_stages_c = [_make_stage_c(c * B2) for c in range(NCHAIN)]


_SC_MESH = plsc.VectorSubcoreMesh(core_axis_name="c", subcore_axis_name="s")


def _gather_max_body(coff, h_hbm, idx_hbm, out_hbm, idx_v, *rest):
    rows = rest[:NRING]
    out_v = rest[NRING]
    sems = rest[NRING + 1:]
    wid = lax.axis_index("s") * 2 + lax.axis_index("c")
    base = wid * PW
    src = h_hbm.at[pl.ds((wid // WPB) * N, N)]
    pltpu.sync_copy(idx_hbm.at[pl.ds(coff + base * K, PW * K)], idx_v)

    def _issue(c, r):
        pltpu.async_copy(src.at[idx_v.at[pl.ds(c * (CH * K), CH * K)]],
                         rows[r], sems[r])

    def _wait(c, r):
        pltpu.make_async_copy(
            src.at[idx_v.at[pl.ds(c * (CH * K), CH * K)]],
            rows[r], sems[r]).wait()

    def _compute(c, r):
        buf = rows[r]
        for i in range(CH):
            for cg in range(C // LG):
                vals = [buf[i * K + k, pl.ds(cg * LG, LG)] for k in range(K)]
                while len(vals) > 1:
                    nxt = [jnp.maximum(vals[j], vals[j + 1])
                           for j in range(0, len(vals) - 1, 2)]
                    if len(vals) % 2:
                        nxt.append(vals[-1])
                    vals = nxt
                out_v[c * CH + i, pl.ds(cg * LG, LG)] = vals[0]

    for r in range(NRING - 1):
        _issue(r, r)

    def _body(g, carry):
        c0 = g * NRING
        for r in range(NRING):
            nxt = c0 + r + NRING - 1

            @pl.when(nxt < NCH)
            def _():
                _issue(nxt, (r + NRING - 1) % NRING)

            _wait(c0 + r, r)
            _compute(c0 + r, r)
        return carry

    lax.fori_loop(0, NCH // NRING, _body, 0)

    pltpu.sync_copy(out_v, out_hbm.at[pl.ds(base, PW)])


def _make_gather_max(coff):
    return pl.kernel(
        functools.partial(_gather_max_body, coff),
        out_type=jax.ShapeDtypeStruct((BN2, C), jnp.bfloat16),
        mesh=_SC_MESH,
        compiler_params=pltpu.CompilerParams(use_tc_tiling_on_sc=False),
        scratch_types=[
            pltpu.VMEM((PW * K,), jnp.int32),
            *[pltpu.VMEM((CH * K, C), jnp.bfloat16) for _ in range(NRING)],
            pltpu.VMEM((PW, C), jnp.bfloat16),
            *[pltpu.SemaphoreType.DMA for _ in range(NRING)],
        ],
    )


_gather_maxes = [_make_gather_max(c * BN2 * K) for c in range(NCHAIN)]


def kernel(xyz, feat, knn_indices, W1, b1, W2, b2, Wm1, bm1, Wm2, bm2, Wl, bl):
    idxf = knn_indices.astype(jnp.int32).reshape(-1)

    W1x = W1[:3]
    W1f = W1[3:]
    W2x = W2[:3]
    W2f = W2[3:]

    ffs, fls = [], []
    for c in range(NCHAIN):
        h1 = _stages_a[c](xyz, feat, W1x, W1f)
        m1 = _gather_maxes[c](h1.reshape(BN2, C), idxf)
        h2 = _stages_b[c](m1.reshape(B2, N, C), xyz, W1x, b1[None, :],
                          W2x, W2f)
        m2 = _gather_maxes[c](h2.reshape(BN2, C), idxf)
        ff, fl = _stages_c[c](m2.reshape(B2, N, C), xyz, W2x, b2[None, :],
                              Wm1, bm1[None, :], Wm2, bm2[:, None],
                              Wl, bl[:, None])
        ffs.append(ff)
        fls.append(fl)
    return (jnp.concatenate(ffs, axis=0), jnp.concatenate(fls, axis=0))


# speedup vs baseline: 1.1465x; 1.0480x over previous
import functools

import jax
import jax.numpy as jnp
from jax import lax
from jax.experimental import pallas as pl
from jax.experimental.pallas import tpu as pltpu, tpu_sc as plsc

B, N, K = 4, 8192, 16
C = 64
B2 = 2
NCHAIN = B // B2
BN2 = B2 * N
NW = 32
PW = BN2 // NW
WPB = N // PW
CH = 8
NCH = PW // CH
NRING = 4
NB = 4096
LG = 32


def _lrelu(x):
    return jnp.where(x >= 0, x, 0.1 * x)


def _dot(a, b, dims):
    return lax.dot_general(a, b, (dims, ((), ())),
                           preferred_element_type=jnp.float32)


def _stage_a_body(xyz_ref, feat_ref, wx_ref, wf_ref, h_ref):
    x = xyz_ref[0]
    f = feat_ref[0]
    h = (_dot(x, wx_ref[...], ((0,), (0,)))
         + _dot(f, wf_ref[...], ((0,), (0,))))
    h_ref[0] = h.astype(jnp.bfloat16)


def _stage_b_body(m_ref, xyz_ref, w1x_ref, b1_ref, w2x_ref, w2f_ref, h2_ref):
    x = xyz_ref[0]
    p1 = _dot(x, w1x_ref[...], ((0,), (0,)))
    f1 = _lrelu(m_ref[0].astype(jnp.float32) - p1 + b1_ref[...])
    h2 = (_dot(x, w2x_ref[...], ((0,), (0,)))
          + _dot(f1, w2f_ref[...], ((1,), (0,))))
    h2_ref[0] = h2.astype(jnp.bfloat16)


def _stage_c_body(m_ref, xyz_ref, w2x_ref, b2_ref, wm1_ref, bm1_ref,
                  wm2_ref, bm2_ref, wl_ref, bl_ref, ff_ref, flow_ref):
    x = xyz_ref[0]
    p2 = _dot(x, w2x_ref[...], ((0,), (0,)))
    f2 = _lrelu(m_ref[0].astype(jnp.float32) - p2 + b2_ref[...])
    f3 = _lrelu(_dot(f2, wm1_ref[...], ((1,), (0,))) + bm1_ref[...])
    ff = _lrelu(_dot(wm2_ref[...], f3, ((0,), (1,))) + bm2_ref[...])
    ff_ref[0] = ff
    flow_ref[0] = _dot(wl_ref[...], ff, ((0,), (0,))) + bl_ref[...]


def _cspec(c, nb, b0):
    return pl.BlockSpec((1, c, nb), lambda b, j: (b0 + b, 0, j))


def _rspec(nb, c, b0=0):
    return pl.BlockSpec((1, nb, c), lambda b, j: (b0 + b, j, 0))


def _fspec(shape):
    return pl.BlockSpec(shape, lambda b, j: tuple(0 for _ in shape))


_GRID = (B2, N // NB)


def _make_stage_a(b0):
    return pl.pallas_call(
        _stage_a_body,
        grid=_GRID,
        in_specs=[_cspec(3, NB, b0), _cspec(C, NB, b0),
                  _fspec((3, C)), _fspec((C, C))],
        out_specs=_rspec(NB, C),
        out_shape=jax.ShapeDtypeStruct((B2, N, C), jnp.bfloat16),
    )


def _make_stage_b(b0):
    return pl.pallas_call(
        _stage_b_body,
        grid=_GRID,
        in_specs=[_rspec(NB, C), _cspec(3, NB, b0), _fspec((3, C)),
                  _fspec((1, C)), _fspec((3, C)), _fspec((C, C))],
        out_specs=_rspec(NB, C),
        out_shape=jax.ShapeDtypeStruct((B2, N, C), jnp.bfloat16),
    )


def _make_stage_c(b0):
    return pl.pallas_call(
        _stage_c_body,
        grid=_GRID,
        in_specs=[_rspec(NB, C), _cspec(3, NB, b0), _fspec((3, C)),
                  _fspec((1, C)), _fspec((C, C)), _fspec((1, C)),
                  _fspec((C, C)), _fspec((C, 1)), _fspec((C, 3)),
                  _fspec((3, 1))],
        out_specs=[pl.BlockSpec((1, C, NB), lambda b, j: (b, 0, j)),
                   pl.BlockSpec((1, 3, NB), lambda b, j: (b, 0, j))],
        out_shape=[jax.ShapeDtypeStruct((B2, C, N), jnp.float32),
                   jax.ShapeDtypeStruct((B2, 3, N), jnp.float32)],
    )


_stages_a = [_make_stage_a(c * B2) for c in range(NCHAIN)]
_stages_b = [_make_stage_b(c * B2) for c in range(NCHAIN)]
_stages_c = [_make_stage_c(c * B2) for c in range(NCHAIN)]


_SC_MESH = plsc.VectorSubcoreMesh(core_axis_name="c", subcore_axis_name="s")


@functools.partial(
    pl.kernel,
    out_type=jax.ShapeDtypeStruct((BN2, C), jnp.bfloat16),
    mesh=_SC_MESH,
    compiler_params=pltpu.CompilerParams(use_tc_tiling_on_sc=False),
    scratch_types=[
        pltpu.VMEM((PW * K,), jnp.int32),
        *[pltpu.VMEM((CH * K, C), jnp.bfloat16) for _ in range(NRING)],
        pltpu.VMEM((PW, C), jnp.bfloat16),
        *[pltpu.SemaphoreType.DMA for _ in range(NRING)],
    ],
)
def _gather_max(h_hbm, idx_hbm, out_hbm, idx_v, *rest):
    rows = rest[:NRING]
    out_v = rest[NRING]
    sems = rest[NRING + 1:]
    wid = lax.axis_index("s") * 2 + lax.axis_index("c")
    base = wid * PW
    src = h_hbm.at[pl.ds((wid // WPB) * N, N)]
    pltpu.sync_copy(idx_hbm.at[pl.ds(base * K, PW * K)], idx_v)

    def _issue(c, r):
        pltpu.async_copy(src.at[idx_v.at[pl.ds(c * (CH * K), CH * K)]],
                         rows[r], sems[r])

    def _wait(c, r):
        pltpu.make_async_copy(
            src.at[idx_v.at[pl.ds(c * (CH * K), CH * K)]],
            rows[r], sems[r]).wait()

    def _compute(c, r):
        buf = rows[r]
        for i in range(CH):
            for cg in range(C // LG):
                vals = [buf[i * K + k, pl.ds(cg * LG, LG)] for k in range(K)]
                while len(vals) > 1:
                    nxt = [jnp.maximum(vals[j], vals[j + 1])
                           for j in range(0, len(vals) - 1, 2)]
                    if len(vals) % 2:
                        nxt.append(vals[-1])
                    vals = nxt
                out_v[c * CH + i, pl.ds(cg * LG, LG)] = vals[0]

    for r in range(NRING - 1):
        _issue(r, r)

    def _body(g, carry):
        c0 = g * NRING
        for r in range(NRING):
            nxt = c0 + r + NRING - 1

            @pl.when(nxt < NCH)
            def _():
                _issue(nxt, (r + NRING - 1) % NRING)

            _wait(c0 + r, r)
            _compute(c0 + r, r)
        return carry

    lax.fori_loop(0, NCH // NRING, _body, 0)

    pltpu.sync_copy(out_v, out_hbm.at[pl.ds(base, PW)])


def kernel(xyz, feat, knn_indices, W1, b1, W2, b2, Wm1, bm1, Wm2, bm2, Wl, bl):
    idx = knn_indices.astype(jnp.int32)

    W1x = W1[:3]
    W1f = W1[3:]
    W2x = W2[:3]
    W2f = W2[3:]

    ffs, fls = [], []
    for c in range(NCHAIN):
        idxf = idx[c * B2:(c + 1) * B2].reshape(-1)
        h1 = _stages_a[c](xyz, feat, W1x, W1f)
        m1 = _gather_max(h1.reshape(BN2, C), idxf)
        h2 = _stages_b[c](m1.reshape(B2, N, C), xyz, W1x, b1[None, :],
                          W2x, W2f)
        m2 = _gather_max(h2.reshape(BN2, C), idxf)
        ff, fl = _stages_c[c](m2.reshape(B2, N, C), xyz, W2x, b2[None, :],
                              Wm1, bm1[None, :], Wm2, bm2[:, None],
                              Wl, bl[:, None])
        ffs.append(ff)
        fls.append(fl)
    return (jnp.concatenate(ffs, axis=0), jnp.concatenate(fls, axis=0))
